# Initial kernel scaffold; baseline (speedup 1.0000x reference)
import jax
import jax.numpy as jnp
from jax.experimental import pallas as pl


def kernel(obs, edges, use_transform_action, num_nodes, rn_mean, rn_var, W_msg0, b_msg0, W_node0, b_node0, W_msg1, b_msg1, W_node1, b_node1, W_msg2, b_msg2, W_node2, b_node2, W_mlp0, b_mlp0, W_mlp1, b_mlp1, W_v, b_v):
    raise NotImplementedError("write your pallas kernel here")


# speedup vs baseline: 4.2705x; 4.2705x over previous
import functools

import jax
import jax.numpy as jnp
from jax import lax
from jax.experimental import pallas as pl
from jax.experimental.pallas import tpu as pltpu
from jax.experimental.pallas import tpu_sc as plsc

N = 10000
E = 320000
B = 8
STATE = 128
MSG = 64
NODE = 128
H0 = 256
H1 = 128

NP = 10240
NTILES = 32
CHUNK = 128
CPT = 80
EP = NTILES * CPT * CHUNK
ROWS_PER_TILE = NP // 16


def _norm_msg_body(obs_ref, mean_ref, var_ref, wm_ref, bm_ref, x_ref, m_ref):
    x = (obs_ref[...] - mean_ref[...]) * lax.rsqrt(var_ref[...] + 1e-8)
    x = jnp.clip(x, -5.0, 5.0)
    x_ref[...] = x
    m = jnp.dot(x, wm_ref[...], preferred_element_type=jnp.float32) + bm_ref[...]
    m_ref[...] = jnp.maximum(m, 0.0)


def _update_msg_body(x_ref, p_ref, wnx_ref, wna_ref, bn_ref, wm_ref, bm_ref,
                     x_out_ref, m_out_ref):
    agg = p_ref[0] + p_ref[1]
    h = (jnp.dot(x_ref[...], wnx_ref[...], preferred_element_type=jnp.float32)
         + jnp.dot(agg, wna_ref[...], preferred_element_type=jnp.float32)
         + bn_ref[...])
    x1 = jnp.maximum(h, 0.0)
    x_out_ref[...] = x1
    m = jnp.dot(x1, wm_ref[...], preferred_element_type=jnp.float32) + bm_ref[...]
    m_out_ref[...] = jnp.maximum(m, 0.0)


def _head_body(x_ref, p0_ref, p1_ref, nn_ref, wnx_ref, wna_ref, bn_ref,
               w0_ref, b0_ref, w1_ref, b1_ref, wv_ref, bv_ref, out_ref):
    xrows = []
    arows = []
    off = 0
    for r in range(B):
        xrows.append(x_ref[pl.ds(off, 1), :])
        arows.append(p0_ref[pl.ds(off, 1), :] + p1_ref[pl.ds(off, 1), :])
        off = off + nn_ref[r]
    xr = jnp.concatenate(xrows, axis=0)
    ar = jnp.concatenate(arows, axis=0)
    h = (jnp.dot(xr, wnx_ref[...], preferred_element_type=jnp.float32)
         + jnp.dot(ar, wna_ref[...], preferred_element_type=jnp.float32)
         + bn_ref[...])
    h = jnp.maximum(h, 0.0)
    h = jnp.tanh(jnp.dot(h, w0_ref[...], preferred_element_type=jnp.float32)
                 + b0_ref[...])
    h = jnp.tanh(jnp.dot(h, w1_ref[...], preferred_element_type=jnp.float32)
                 + b1_ref[...])
    v = jnp.sum(h * wv_ref[...], axis=1, keepdims=True) + bv_ref[...]
    out_ref[...] = v


def _seg_sum_body(m_hbm, srcd_hbm, dstd_hbm, p_hbm,
                  src_v, dst_v, rows_v, zbuf_v, acc_sh, sem):
    cid = lax.axis_index("c")
    sid = lax.axis_index("s")
    wid = sid * 2 + cid

    zv = jnp.zeros((16,), jnp.float32)
    for i in range(16):
        for j in range(MSG // 16):
            zbuf_v[i, pl.ds(j * 16, 16)] = zv

    def zero_body(k, _):
        pltpu.sync_copy(zbuf_v, acc_sh.at[pl.ds(sid * ROWS_PER_TILE + k * 16, 16)])
        return 0
    lax.fori_loop(0, ROWS_PER_TILE // 16, zero_body, 0)

    pltpu.sync_copy(srcd_hbm.at[pl.ds(wid * CPT, CPT)], src_v)
    pltpu.sync_copy(dstd_hbm.at[pl.ds(wid * CPT, CPT)], dst_v)

    plsc.subcore_barrier()

    def chunk_body(j, _):
        pltpu.async_copy(m_hbm.at[src_v.at[j]], rows_v, sem).wait()
        pltpu.sync_copy(rows_v, acc_sh.at[dst_v.at[j]], add=True)
        return 0
    lax.fori_loop(0, CPT, chunk_body, 0)

    plsc.subcore_barrier()

    pltpu.sync_copy(acc_sh.at[pl.ds(sid * ROWS_PER_TILE, ROWS_PER_TILE)],
                    p_hbm.at[cid, pl.ds(sid * ROWS_PER_TILE, ROWS_PER_TILE)])


def _make_seg_sum():
    mesh = plsc.VectorSubcoreMesh(core_axis_name="c", subcore_axis_name="s",
                                  num_cores=2, num_subcores=16)
    return pl.kernel(
        _seg_sum_body,
        out_type=jax.ShapeDtypeStruct((2, NP, MSG), jnp.float32),
        mesh=mesh,
        scratch_types=[
            pltpu.VMEM((CPT, CHUNK), jnp.int32),
            pltpu.VMEM((CPT, CHUNK), jnp.int32),
            pltpu.VMEM((CHUNK, MSG), jnp.float32),
            pltpu.VMEM((16, MSG), jnp.float32),
            pltpu.VMEM_SHARED((NP, MSG), jnp.float32),
            pltpu.SemaphoreType.DMA,
        ],
        compiler_params=pltpu.CompilerParams(use_tc_tiling_on_sc=False),
    )


_GRID = 10
_BLK = NP // _GRID


def _row_spec(width):
    return pl.BlockSpec((_BLK, width), lambda i: (i, 0))


def _full_spec(a, b):
    return pl.BlockSpec((a, b), lambda i: (0, 0))


def kernel(obs, edges, use_transform_action, num_nodes, rn_mean, rn_var,
           W_msg0, b_msg0, W_node0, b_node0,
           W_msg1, b_msg1, W_node1, b_node1,
           W_msg2, b_msg2, W_node2, b_node2,
           W_mlp0, b_mlp0, W_mlp1, b_mlp1, W_v, b_v):
    f32 = jnp.float32
    obs_p = jnp.pad(obs, ((0, NP - N), (0, 0)))
    src = jnp.concatenate([edges[0], jnp.full((EP - E,), NP - 1, jnp.int32)])
    dst = jnp.concatenate([edges[1], jnp.full((EP - E,), NP - 1, jnp.int32)])
    srcd = src.reshape(NTILES * CPT, CHUNK)
    dstd = dst.reshape(NTILES * CPT, CHUNK)

    mean2 = rn_mean.reshape(1, STATE)
    var2 = rn_var.reshape(1, STATE)
    msg_w = [(W_msg0, b_msg0.reshape(1, MSG)),
             (W_msg1, b_msg1.reshape(1, MSG)),
             (W_msg2, b_msg2.reshape(1, MSG))]
    node_w = [(W_node0[:NODE], W_node0[NODE:], b_node0.reshape(1, NODE)),
              (W_node1[:NODE], W_node1[NODE:], b_node1.reshape(1, NODE)),
              (W_node2[:NODE], W_node2[NODE:], b_node2.reshape(1, NODE))]

    norm_msg = pl.pallas_call(
        _norm_msg_body,
        grid=(_GRID,),
        in_specs=[_row_spec(STATE), _full_spec(1, STATE), _full_spec(1, STATE),
                  _full_spec(NODE, MSG), _full_spec(1, MSG)],
        out_specs=[_row_spec(NODE), _row_spec(MSG)],
        out_shape=[jax.ShapeDtypeStruct((NP, NODE), f32),
                   jax.ShapeDtypeStruct((NP, MSG), f32)],
    )
    x0, m0 = norm_msg(obs_p, mean2, var2, msg_w[0][0], msg_w[0][1])

    seg_sum = _make_seg_sum()

    update_msg = pl.pallas_call(
        _update_msg_body,
        grid=(_GRID,),
        in_specs=[_row_spec(NODE),
                  pl.BlockSpec((2, _BLK, MSG), lambda i: (0, i, 0)),
                  _full_spec(NODE, NODE), _full_spec(MSG, NODE),
                  _full_spec(1, NODE),
                  _full_spec(NODE, MSG), _full_spec(1, MSG)],
        out_specs=[_row_spec(NODE), _row_spec(MSG)],
        out_shape=[jax.ShapeDtypeStruct((NP, NODE), f32),
                   jax.ShapeDtypeStruct((NP, MSG), f32)],
    )

    x, m = x0, m0
    for i in range(2):
        p = seg_sum(m, srcd, dstd)
        wnx, wna, bn = node_w[i]
        x, m = update_msg(x, p, wnx, wna, bn, msg_w[i + 1][0], msg_w[i + 1][1])

    p2 = seg_sum(m, srcd, dstd)

    wnx2, wna2, bn2 = node_w[2]
    head = pl.pallas_call(
        _head_body,
        in_specs=[pl.BlockSpec(memory_space=pltpu.VMEM),
                  pl.BlockSpec(memory_space=pltpu.VMEM),
                  pl.BlockSpec(memory_space=pltpu.VMEM),
                  pl.BlockSpec(memory_space=pltpu.SMEM),
                  pl.BlockSpec(memory_space=pltpu.VMEM),
                  pl.BlockSpec(memory_space=pltpu.VMEM),
                  pl.BlockSpec(memory_space=pltpu.VMEM),
                  pl.BlockSpec(memory_space=pltpu.VMEM),
                  pl.BlockSpec(memory_space=pltpu.VMEM),
                  pl.BlockSpec(memory_space=pltpu.VMEM),
                  pl.BlockSpec(memory_space=pltpu.VMEM),
                  pl.BlockSpec(memory_space=pltpu.VMEM),
                  pl.BlockSpec(memory_space=pltpu.VMEM)],
        out_shape=jax.ShapeDtypeStruct((B, 1), f32),
    )
    value = head(x, p2[0], p2[1], num_nodes, wnx2, wna2, bn2,
                 W_mlp0, b_mlp0.reshape(1, H0), W_mlp1, b_mlp1.reshape(1, H1),
                 W_v.reshape(1, NODE), b_v.reshape(1, 1))
    return value


# speedup vs baseline: 5.0300x; 1.1779x over previous
import functools

import jax
import jax.numpy as jnp
from jax import lax
from jax.experimental import pallas as pl
from jax.experimental.pallas import tpu as pltpu
from jax.experimental.pallas import tpu_sc as plsc

N = 10000
E = 320000
B = 8
STATE = 128
MSG = 64
NODE = 128
H0 = 256
H1 = 128

NP = 10240
NTILES = 32
CHUNK = 128
CPT = 80
EP = NTILES * CPT * CHUNK
ROWS_PER_TILE = NP // 16


def _norm_msg_body(obs_ref, mean_ref, var_ref, wm_ref, bm_ref, x_ref, m_ref):
    x = (obs_ref[...] - mean_ref[...]) * lax.rsqrt(var_ref[...] + 1e-8)
    x = jnp.clip(x, -5.0, 5.0)
    x_ref[...] = x
    m = jnp.dot(x, wm_ref[...], preferred_element_type=jnp.float32) + bm_ref[...]
    m_ref[...] = jnp.maximum(m, 0.0)


def _update_msg_body(x_ref, p_ref, wnx_ref, wna_ref, bn_ref, wm_ref, bm_ref,
                     x_out_ref, m_out_ref):
    agg = p_ref[0] + p_ref[1]
    h = (jnp.dot(x_ref[...], wnx_ref[...], preferred_element_type=jnp.float32)
         + jnp.dot(agg, wna_ref[...], preferred_element_type=jnp.float32)
         + bn_ref[...])
    x1 = jnp.maximum(h, 0.0)
    x_out_ref[...] = x1
    m = jnp.dot(x1, wm_ref[...], preferred_element_type=jnp.float32) + bm_ref[...]
    m_out_ref[...] = jnp.maximum(m, 0.0)


def _head_body(x_ref, p0_ref, p1_ref, nn_ref, wnx_ref, wna_ref, bn_ref,
               w0_ref, b0_ref, w1_ref, b1_ref, wv_ref, bv_ref, out_ref):
    xrows = []
    arows = []
    off = 0
    for r in range(B):
        xrows.append(x_ref[pl.ds(off, 1), :])
        arows.append(p0_ref[pl.ds(off, 1), :] + p1_ref[pl.ds(off, 1), :])
        off = off + nn_ref[r]
    xr = jnp.concatenate(xrows, axis=0)
    ar = jnp.concatenate(arows, axis=0)
    h = (jnp.dot(xr, wnx_ref[...], preferred_element_type=jnp.float32)
         + jnp.dot(ar, wna_ref[...], preferred_element_type=jnp.float32)
         + bn_ref[...])
    h = jnp.maximum(h, 0.0)
    h = jnp.tanh(jnp.dot(h, w0_ref[...], preferred_element_type=jnp.float32)
                 + b0_ref[...])
    h = jnp.tanh(jnp.dot(h, w1_ref[...], preferred_element_type=jnp.float32)
                 + b1_ref[...])
    v = jnp.sum(h * wv_ref[...], axis=1, keepdims=True) + bv_ref[...]
    out_ref[...] = v


def _seg_sum_body(m_hbm, srcd_hbm, dstd_hbm, p_hbm,
                  src_v, dst_v, rows_v, zbuf_v, acc_sh, sem):
    cid = lax.axis_index("c")
    sid = lax.axis_index("s")
    wid = sid * 2 + cid

    zv = jnp.zeros((16,), jnp.float32)
    for i in range(16):
        for j in range(MSG // 16):
            zbuf_v[i, pl.ds(j * 16, 16)] = zv

    def zero_body(k, _):
        pltpu.sync_copy(zbuf_v, acc_sh.at[pl.ds(sid * ROWS_PER_TILE + k * 16, 16)])
        return 0
    lax.fori_loop(0, ROWS_PER_TILE // 16, zero_body, 0)

    pltpu.sync_copy(srcd_hbm.at[pl.ds(wid * CPT, CPT)], src_v)
    pltpu.sync_copy(dstd_hbm.at[pl.ds(wid * CPT, CPT)], dst_v)

    plsc.subcore_barrier()

    rows_a, rows_b = rows_v.at[0], rows_v.at[1]
    sem_a, sem_b = sem.at[0], sem.at[1]
    pltpu.async_copy(m_hbm.at[src_v.at[0]], rows_a, sem_a)

    def chunk_body(jj, _):
        j0 = jj * 2
        j1 = j0 + 1
        pltpu.async_copy(m_hbm.at[src_v.at[j1]], rows_b, sem_b)
        pltpu.make_async_copy(m_hbm.at[src_v.at[j0]], rows_a, sem_a).wait()
        pltpu.sync_copy(rows_a, acc_sh.at[dst_v.at[j0]], add=True)
        jn = jnp.minimum(j0 + 2, CPT - 1)
        pltpu.async_copy(m_hbm.at[src_v.at[jn]], rows_a, sem_a)
        pltpu.make_async_copy(m_hbm.at[src_v.at[j1]], rows_b, sem_b).wait()
        pltpu.sync_copy(rows_b, acc_sh.at[dst_v.at[j1]], add=True)
        return 0
    lax.fori_loop(0, CPT // 2, chunk_body, 0)
    pltpu.make_async_copy(m_hbm.at[src_v.at[0]], rows_a, sem_a).wait()

    plsc.subcore_barrier()

    pltpu.sync_copy(acc_sh.at[pl.ds(sid * ROWS_PER_TILE, ROWS_PER_TILE)],
                    p_hbm.at[cid, pl.ds(sid * ROWS_PER_TILE, ROWS_PER_TILE)])


def _make_seg_sum():
    mesh = plsc.VectorSubcoreMesh(core_axis_name="c", subcore_axis_name="s",
                                  num_cores=2, num_subcores=16)
    return pl.kernel(
        _seg_sum_body,
        out_type=jax.ShapeDtypeStruct((2, NP, MSG), jnp.float32),
        mesh=mesh,
        scratch_types=[
            pltpu.VMEM((CPT, CHUNK), jnp.int32),
            pltpu.VMEM((CPT, CHUNK), jnp.int32),
            pltpu.VMEM((2, CHUNK, MSG), jnp.float32),
            pltpu.VMEM((16, MSG), jnp.float32),
            pltpu.VMEM_SHARED((NP, MSG), jnp.float32),
            pltpu.SemaphoreType.DMA((2,)),
        ],
        compiler_params=pltpu.CompilerParams(use_tc_tiling_on_sc=False),
    )


_GRID = 10
_BLK = NP // _GRID


def _row_spec(width):
    return pl.BlockSpec((_BLK, width), lambda i: (i, 0))


def _full_spec(a, b):
    return pl.BlockSpec((a, b), lambda i: (0, 0))


def kernel(obs, edges, use_transform_action, num_nodes, rn_mean, rn_var,
           W_msg0, b_msg0, W_node0, b_node0,
           W_msg1, b_msg1, W_node1, b_node1,
           W_msg2, b_msg2, W_node2, b_node2,
           W_mlp0, b_mlp0, W_mlp1, b_mlp1, W_v, b_v):
    f32 = jnp.float32
    obs_p = jnp.pad(obs, ((0, NP - N), (0, 0)))
    src = jnp.concatenate([edges[0], jnp.full((EP - E,), NP - 1, jnp.int32)])
    dst = jnp.concatenate([edges[1], jnp.full((EP - E,), NP - 1, jnp.int32)])
    srcd = src.reshape(NTILES * CPT, CHUNK)
    dstd = dst.reshape(NTILES * CPT, CHUNK)

    mean2 = rn_mean.reshape(1, STATE)
    var2 = rn_var.reshape(1, STATE)
    msg_w = [(W_msg0, b_msg0.reshape(1, MSG)),
             (W_msg1, b_msg1.reshape(1, MSG)),
             (W_msg2, b_msg2.reshape(1, MSG))]
    node_w = [(W_node0[:NODE], W_node0[NODE:], b_node0.reshape(1, NODE)),
              (W_node1[:NODE], W_node1[NODE:], b_node1.reshape(1, NODE)),
              (W_node2[:NODE], W_node2[NODE:], b_node2.reshape(1, NODE))]

    norm_msg = pl.pallas_call(
        _norm_msg_body,
        grid=(_GRID,),
        in_specs=[_row_spec(STATE), _full_spec(1, STATE), _full_spec(1, STATE),
                  _full_spec(NODE, MSG), _full_spec(1, MSG)],
        out_specs=[_row_spec(NODE), _row_spec(MSG)],
        out_shape=[jax.ShapeDtypeStruct((NP, NODE), f32),
                   jax.ShapeDtypeStruct((NP, MSG), f32)],
    )
    x0, m0 = norm_msg(obs_p, mean2, var2, msg_w[0][0], msg_w[0][1])

    seg_sum = _make_seg_sum()

    update_msg = pl.pallas_call(
        _update_msg_body,
        grid=(_GRID,),
        in_specs=[_row_spec(NODE),
                  pl.BlockSpec((2, _BLK, MSG), lambda i: (0, i, 0)),
                  _full_spec(NODE, NODE), _full_spec(MSG, NODE),
                  _full_spec(1, NODE),
                  _full_spec(NODE, MSG), _full_spec(1, MSG)],
        out_specs=[_row_spec(NODE), _row_spec(MSG)],
        out_shape=[jax.ShapeDtypeStruct((NP, NODE), f32),
                   jax.ShapeDtypeStruct((NP, MSG), f32)],
    )

    x, m = x0, m0
    for i in range(2):
        p = seg_sum(m, srcd, dstd)
        wnx, wna, bn = node_w[i]
        x, m = update_msg(x, p, wnx, wna, bn, msg_w[i + 1][0], msg_w[i + 1][1])

    p2 = seg_sum(m, srcd, dstd)

    wnx2, wna2, bn2 = node_w[2]
    head = pl.pallas_call(
        _head_body,
        in_specs=[pl.BlockSpec(memory_space=pltpu.VMEM),
                  pl.BlockSpec(memory_space=pltpu.VMEM),
                  pl.BlockSpec(memory_space=pltpu.VMEM),
                  pl.BlockSpec(memory_space=pltpu.SMEM),
                  pl.BlockSpec(memory_space=pltpu.VMEM),
                  pl.BlockSpec(memory_space=pltpu.VMEM),
                  pl.BlockSpec(memory_space=pltpu.VMEM),
                  pl.BlockSpec(memory_space=pltpu.VMEM),
                  pl.BlockSpec(memory_space=pltpu.VMEM),
                  pl.BlockSpec(memory_space=pltpu.VMEM),
                  pl.BlockSpec(memory_space=pltpu.VMEM),
                  pl.BlockSpec(memory_space=pltpu.VMEM),
                  pl.BlockSpec(memory_space=pltpu.VMEM)],
        out_shape=jax.ShapeDtypeStruct((B, 1), f32),
    )
    value = head(x, p2[0], p2[1], num_nodes, wnx2, wna2, bn2,
                 W_mlp0, b_mlp0.reshape(1, H0), W_mlp1, b_mlp1.reshape(1, H1),
                 W_v.reshape(1, NODE), b_v.reshape(1, 1))
    return value
